# baseline (speedup 1.0000x reference)
import functools

import jax
import jax.numpy as jnp
from jax import lax
from jax.experimental import pallas as pl

MAX_BOXES_OUT = 5000
FILL = -1
LANE = 128
ROWS_PER_BLOCK = 32
BW_LABELS = 1024


@functools.partial(jax.jit, static_argnames=("b", "n", "d", "m"))
def _pad_dense(boxes3, labels, b, n, d, m):
    ldtype = labels.dtype
    nb = n * d
    rin = boxes3.shape[1]
    rout_total = (m * d + LANE - 1) // LANE
    rout = ((rout_total + ROWS_PER_BLOCK - 1) // ROWS_PER_BLOCK
            ) * ROWS_PER_BLOCK
    grid = rout // ROWS_PER_BLOCK
    in_blocks = rin // ROWS_PER_BLOCK
    in_blocks_l = pl.cdiv(n, BW_LABELS)

    def body(b_ref, l_ref, ob_ref, ol_ref):
        i = pl.program_id(0)
        pos = (i * ROWS_PER_BLOCK * LANE
               + lax.broadcasted_iota(jnp.int32, (b, ROWS_PER_BLOCK, LANE), 1)
               * LANE
               + lax.broadcasted_iota(jnp.int32, (b, ROWS_PER_BLOCK, LANE), 2))
        ob_ref[...] = jnp.where(pos < nb, b_ref[...],
                                jnp.float32(FILL).astype(boxes3.dtype))
        coll = i * BW_LABELS + lax.broadcasted_iota(
            jnp.int32, (b, BW_LABELS), 1)
        ol_ref[...] = jnp.where(coll < n, l_ref[...], jnp.array(FILL, ldtype))

    return pl.pallas_call(
        body,
        grid=(grid,),
        in_specs=[
            pl.BlockSpec((b, ROWS_PER_BLOCK, LANE),
                         lambda i: (0, jnp.minimum(i, in_blocks - 1), 0)),
            pl.BlockSpec((b, BW_LABELS),
                         lambda i: (0, jnp.minimum(i, in_blocks_l - 1))),
        ],
        out_specs=[
            pl.BlockSpec((b, ROWS_PER_BLOCK, LANE), lambda i: (0, i, 0)),
            pl.BlockSpec((b, BW_LABELS), lambda i: (0, i)),
        ],
        out_shape=[
            jax.ShapeDtypeStruct((b, rout, LANE), boxes3.dtype),
            jax.ShapeDtypeStruct((b, m), ldtype),
        ],
    )(boxes3, labels)


def kernel(boxes, labels):
    b, n, d = boxes.shape
    m = MAX_BOXES_OUT
    n_pad = ((n + LANE - 1) // LANE) * LANE
    boxes_p = jnp.concatenate(
        [boxes, jnp.zeros((b, n_pad - n, d), boxes.dtype)], axis=1)
    boxes3 = boxes_p.reshape(b, (n_pad * d) // LANE, LANE)
    bo3, labels_out = _pad_dense(boxes3, labels, b, n, d, m)
    m_pad = (bo3.shape[1] * LANE) // d
    boxes_out = bo3.reshape(b, m_pad, d)[:, :m, :]
    return boxes_out, labels_out


# speedup vs baseline: 1.4761x; 1.4761x over previous
import functools

import jax
import jax.numpy as jnp
from jax import lax
from jax.experimental import pallas as pl

MAX_BOXES_OUT = 5000
FILL = -1
BW_BOXES = 4096
BW_LABELS = 1024


@functools.partial(jax.jit, static_argnames=("b", "n", "d", "m"))
def _pad_dense(boxes2, labels, b, n, d, m):
    ldtype = labels.dtype
    nb = n * d
    mb = m * d
    gb = pl.cdiv(mb, BW_BOXES)
    gl = pl.cdiv(m, BW_LABELS)
    grid = max(gb, gl)
    in_blocks_b = pl.cdiv(nb, BW_BOXES)
    in_blocks_l = pl.cdiv(n, BW_LABELS)

    def body(b_ref, l_ref, ob_ref, ol_ref):
        i = pl.program_id(0)
        colb = i * BW_BOXES + lax.broadcasted_iota(jnp.int32, (b, BW_BOXES), 1)
        ob_ref[...] = jnp.where(colb < nb, b_ref[...],
                                jnp.float32(FILL).astype(boxes2.dtype))
        coll = i * BW_LABELS + lax.broadcasted_iota(jnp.int32, (b, BW_LABELS), 1)
        ol_ref[...] = jnp.where(coll < n, l_ref[...],
                                jnp.array(FILL, ldtype))

    return pl.pallas_call(
        body,
        grid=(grid,),
        in_specs=[
            pl.BlockSpec((b, BW_BOXES),
                         lambda i: (0, jnp.minimum(i, in_blocks_b - 1))),
            pl.BlockSpec((b, BW_LABELS),
                         lambda i: (0, jnp.minimum(i, in_blocks_l - 1))),
        ],
        out_specs=[
            pl.BlockSpec((b, BW_BOXES), lambda i: (0, i)),
            pl.BlockSpec((b, BW_LABELS), lambda i: (0, i)),
        ],
        out_shape=[
            jax.ShapeDtypeStruct((b, mb), boxes2.dtype),
            jax.ShapeDtypeStruct((b, m), ldtype),
        ],
    )(boxes2, labels)


def kernel(boxes, labels):
    b, n, d = boxes.shape
    m = MAX_BOXES_OUT
    boxes_out2, labels_out = _pad_dense(boxes.reshape(b, n * d), labels,
                                        b, n, d, m)
    return boxes_out2.reshape(b, m, d), labels_out
